# Initial kernel scaffold; baseline (speedup 1.0000x reference)
import jax
import jax.numpy as jnp
from jax.experimental import pallas as pl


def kernel(embedding, memory_bank):
    raise NotImplementedError("write your pallas kernel here")


# speedup vs baseline: 8.1760x; 8.1760x over previous
import functools

import jax
import jax.numpy as jnp
from jax.experimental import pallas as pl
from jax.experimental.pallas import tpu as pltpu

BATCH = 8
NUM_PATCHES = 784
D = 128
M = 16384
K_NN = 9
CHUNK = 2048
NUM_CHUNKS = M // CHUNK


def _phase_a_kernel(emb_ref, mb_ref, feat_ref, score_ref, nnidx_ref):
    x = emb_ref[...]
    x2 = jnp.sum(x * x, axis=1)

    def body(c, carry):
        run_min, run_idx = carry
        chunk = mb_ref[pl.ds(c * CHUNK, CHUNK), :]
        mb2 = jnp.sum(chunk * chunk, axis=1, keepdims=True)
        xy = jax.lax.dot_general(
            chunk, x, (((1,), (1,)), ((), ())),
            preferred_element_type=jnp.float32)
        s = mb2 - 2.0 * xy
        mn = jnp.min(s, axis=0, keepdims=True)
        ridx = jax.lax.broadcasted_iota(jnp.int32, s.shape, 0) + c * CHUNK
        am = jnp.min(jnp.where(s == mn, ridx, M), axis=0, keepdims=True)
        better = mn < run_min
        return jnp.minimum(run_min, mn), jnp.where(better, am, run_idx)

    init = (jnp.full((1, NUM_PATCHES), jnp.inf, jnp.float32),
            jnp.zeros((1, NUM_PATCHES), jnp.int32))
    smin, sidx = jax.lax.fori_loop(0, NUM_CHUNKS, body, init)

    mind2 = smin + x2.reshape(1, NUM_PATCHES)
    p = jnp.argmax(mind2)
    score = jnp.sqrt(jnp.maximum(jnp.max(mind2), 1e-12))
    lane = jax.lax.broadcasted_iota(jnp.int32, (1, NUM_PATCHES), 1)
    nn_idx = jnp.max(jnp.where(lane == p, sidx, -1))

    feat_ref[...] = emb_ref[pl.ds(p, 1), :].reshape(1, 1, D)
    score_ref[...] = jnp.full((1, 1, D), score, jnp.float32)
    nnidx_ref[...] = jnp.full((1, 1, D), nn_idx, jnp.int32)


def _phase_b_kernel(mb_ref, feat_ref, scorevec_ref, nnidx_ref, out_ref):
    mb = mb_ref[...]
    feat = feat_ref[...]
    mb2 = jnp.sum(mb * mb, axis=1, keepdims=True)

    ns = jnp.concatenate(
        [mb_ref[pl.ds(nnidx_ref[b], 1), :] for b in range(BATCH)], axis=0)

    g = jax.lax.dot_general(mb, ns, (((1,), (1,)), ((), ())),
                            preferred_element_type=jnp.float32)
    s = mb2 - 2.0 * g
    f = jax.lax.dot_general(mb, feat, (((1,), (1,)), ((), ())),
                            preferred_element_type=jnp.float32)
    dpart = mb2 - 2.0 * f

    ridx = jax.lax.broadcasted_iota(jnp.int32, (M, BATCH), 0)
    vals = []
    for _ in range(K_NN):
        mn = jnp.min(s, axis=0, keepdims=True)
        am = jnp.min(jnp.where(s == mn, ridx, M), axis=0, keepdims=True)
        mask = ridx == am
        vals.append(jnp.sum(jnp.where(mask, dpart, 0.0), axis=0))
        s = jnp.where(mask, jnp.inf, s)

    v = jnp.stack(vals, axis=0)
    f2 = jnp.sum(feat * feat, axis=1).reshape(1, BATCH)
    d3 = jnp.sqrt(jnp.maximum(v + f2, 1e-12))
    mx = jnp.max(d3, axis=0, keepdims=True)
    e = jnp.exp(d3 - mx)
    w0 = 1.0 - e[0:1, :] / jnp.sum(e, axis=0, keepdims=True)
    out_ref[...] = w0 * scorevec_ref[...]


@jax.jit
def kernel(embedding, memory_bank):
    feat, scoreb, nnidxb = pl.pallas_call(
        _phase_a_kernel,
        grid=(BATCH,),
        in_specs=[
            pl.BlockSpec((NUM_PATCHES, D), lambda b: (b, 0)),
            pl.BlockSpec((M, D), lambda b: (0, 0)),
        ],
        out_specs=[
            pl.BlockSpec((1, 1, D), lambda b: (b, 0, 0)),
            pl.BlockSpec((1, 1, D), lambda b: (b, 0, 0)),
            pl.BlockSpec((1, 1, D), lambda b: (b, 0, 0)),
        ],
        out_shape=[
            jax.ShapeDtypeStruct((BATCH, 1, D), jnp.float32),
            jax.ShapeDtypeStruct((BATCH, 1, D), jnp.float32),
            jax.ShapeDtypeStruct((BATCH, 1, D), jnp.int32),
        ],
    )(embedding, memory_bank)

    feat2d = feat.reshape(BATCH, D)
    scorevec = scoreb[:, 0, 0].reshape(1, BATCH)
    nnidx = nnidxb[:, 0, 0]

    pred = pl.pallas_call(
        _phase_b_kernel,
        in_specs=[
            pl.BlockSpec(memory_space=pltpu.VMEM),
            pl.BlockSpec(memory_space=pltpu.VMEM),
            pl.BlockSpec(memory_space=pltpu.VMEM),
            pl.BlockSpec(memory_space=pltpu.SMEM),
        ],
        out_shape=jax.ShapeDtypeStruct((1, BATCH), jnp.float32),
    )(memory_bank, feat2d, scorevec, nnidx)
    return pred.reshape(BATCH)


# speedup vs baseline: 10.2733x; 1.2565x over previous
import jax
import jax.numpy as jnp
from jax.experimental import pallas as pl
from jax.experimental.pallas import tpu as pltpu

BATCH = 8
NUM_PATCHES = 784
D = 128
M = 16384
K_NN = 9
CHUNK = 2048
NUM_CHUNKS = M // CHUNK


def _phase_a_kernel(emb_ref, mb_ref, feat_ref, score_ref, nnidx_ref, mb2_ref):
    b = pl.program_id(0)

    @pl.when(b == 0)
    def _():
        mb = mb_ref[...]
        mb2_ref[...] = jnp.sum(mb * mb, axis=1, keepdims=True)

    x = emb_ref[...]
    x2 = jnp.sum(x * x, axis=1)

    def body(c, run_min):
        chunk = mb_ref[pl.ds(c * CHUNK, CHUNK), :]
        mb2 = mb2_ref[pl.ds(c * CHUNK, CHUNK), :]
        xy = jax.lax.dot_general(
            chunk, x, (((1,), (1,)), ((), ())),
            preferred_element_type=jnp.float32)
        s = mb2 - 2.0 * xy
        return jnp.minimum(run_min, jnp.min(s, axis=0, keepdims=True))

    init = jnp.full((1, NUM_PATCHES), jnp.inf, jnp.float32)
    smin = jax.lax.fori_loop(0, NUM_CHUNKS, body, init)

    mind2 = smin + x2.reshape(1, NUM_PATCHES)
    p = jnp.argmax(mind2)
    score = jnp.sqrt(jnp.maximum(jnp.max(mind2), 1e-12))
    feat = emb_ref[pl.ds(p, 1), :]

    w = jax.lax.dot_general(mb_ref[...], feat, (((1,), (1,)), ((), ())),
                            preferred_element_type=jnp.float32)
    sw = mb2_ref[...] - 2.0 * w
    mnw = jnp.min(sw)
    ridx = jax.lax.broadcasted_iota(jnp.int32, (M, 1), 0)
    nn_idx = jnp.min(jnp.where(sw == mnw, ridx, M))

    feat_ref[...] = feat.reshape(1, 1, D)
    score_ref[...] = jnp.full((1, 1, D), score, jnp.float32)
    nnidx_ref[...] = jnp.full((1, 1, D), nn_idx, jnp.int32)


def _phase_b_kernel(mb_ref, feat_ref, score_ref, nnidx_ref, out_ref):
    mb = mb_ref[...]
    feat = feat_ref[...]
    mb2 = jnp.sum(mb * mb, axis=1, keepdims=True)

    ns = jnp.concatenate(
        [mb_ref[pl.ds(nnidx_ref[b], 1), :] for b in range(BATCH)], axis=0)

    g = jax.lax.dot_general(mb, ns, (((1,), (1,)), ((), ())),
                            preferred_element_type=jnp.float32)
    s = mb2 - 2.0 * g
    f = jax.lax.dot_general(mb, feat, (((1,), (1,)), ((), ())),
                            preferred_element_type=jnp.float32)
    dpart = mb2 - 2.0 * f

    ridx = jax.lax.broadcasted_iota(jnp.int32, (M, BATCH), 0)
    vals = []
    for _ in range(K_NN):
        mn = jnp.min(s, axis=0, keepdims=True)
        am = jnp.min(jnp.where(s == mn, ridx, M), axis=0, keepdims=True)
        mask = ridx == am
        vals.append(jnp.sum(jnp.where(mask, dpart, 0.0), axis=0, keepdims=True))
        s = jnp.where(mask, jnp.inf, s)

    v = jnp.concatenate(vals, axis=0)
    f2 = jnp.sum(feat * feat, axis=1).reshape(1, BATCH)
    d3 = jnp.sqrt(jnp.maximum(v + f2, 1e-12))
    mx = jnp.max(d3, axis=0, keepdims=True)
    e = jnp.exp(d3 - mx)
    w0 = 1.0 - e[0:1, :] / jnp.sum(e, axis=0, keepdims=True)
    out_ref[...] = w0 * score_ref[...]


@jax.jit
def kernel(embedding, memory_bank):
    feat, scoreb, nnidxb = pl.pallas_call(
        _phase_a_kernel,
        grid=(BATCH,),
        in_specs=[
            pl.BlockSpec((NUM_PATCHES, D), lambda b: (b, 0)),
            pl.BlockSpec((M, D), lambda b: (0, 0)),
        ],
        out_specs=[
            pl.BlockSpec((1, 1, D), lambda b: (b, 0, 0)),
            pl.BlockSpec((1, 1, D), lambda b: (b, 0, 0)),
            pl.BlockSpec((1, 1, D), lambda b: (b, 0, 0)),
        ],
        out_shape=[
            jax.ShapeDtypeStruct((BATCH, 1, D), jnp.float32),
            jax.ShapeDtypeStruct((BATCH, 1, D), jnp.float32),
            jax.ShapeDtypeStruct((BATCH, 1, D), jnp.int32),
        ],
        scratch_shapes=[pltpu.VMEM((M, 1), jnp.float32)],
    )(embedding, memory_bank)

    feat2d = feat.reshape(BATCH, D)
    scorerow = scoreb[:, 0, 0].reshape(1, BATCH)
    nnidx = nnidxb[:, 0, 0]

    pred = pl.pallas_call(
        _phase_b_kernel,
        in_specs=[
            pl.BlockSpec(memory_space=pltpu.VMEM),
            pl.BlockSpec(memory_space=pltpu.VMEM),
            pl.BlockSpec(memory_space=pltpu.VMEM),
            pl.BlockSpec(memory_space=pltpu.SMEM),
        ],
        out_shape=jax.ShapeDtypeStruct((1, BATCH), jnp.float32),
    )(memory_bank, feat2d, scorerow, nnidx)
    return pred.reshape(BATCH)
